# baseline (speedup 1.0000x reference)
import functools

import jax
import jax.numpy as jnp
from jax import lax
from jax.experimental import pallas as pl
from jax.experimental.pallas import tpu as pltpu
from jax.experimental.pallas import tpu_sc as plsc

B = 16384
MF_D = 16
MLP_D = 10
VROWS = 625000
NC = 2
NS = 16
NW = NC * NS
BPW = B // NW


@functools.cache
def _make_sc_gather():
    mesh = plsc.VectorSubcoreMesh(core_axis_name="c", subcore_axis_name="s")

    @functools.partial(
        pl.kernel,
        mesh=mesh,
        compiler_params=pltpu.CompilerParams(use_tc_tiling_on_sc=False),
        out_type=[jax.ShapeDtypeStruct((B, 16), jnp.float32)] * 6,
        scratch_types=(
            [pltpu.VMEM((BPW,), jnp.int32)] * 6
            + [pltpu.VMEM((BPW, 16), jnp.float32)] * 6
            + [pltpu.SemaphoreType.DMA]
        ),
    )
    def _sc_gather(u_hbm, i_hbm, au_hbm, au2_hbm, ai_hbm, ai2_hbm,
                   mfu_hbm, mfi_hbm, mlpu_hbm, mlpi_hbm,
                   o_mfu, o_mfi, o_u0, o_u1, o_i0, o_i1,
                   uv, iv, auv, au2v, aiv, ai2v,
                   bmfu, bmfi, bu0, bu1, bi0, bi1, sem):
        wid = lax.axis_index("s") * NC + lax.axis_index("c")
        base = wid * BPW
        pltpu.sync_copy(u_hbm.at[pl.ds(base, BPW)], uv)
        pltpu.sync_copy(i_hbm.at[pl.ds(base, BPW)], iv)
        pltpu.sync_copy(au_hbm.at[pl.ds(base, BPW)], auv)
        pltpu.sync_copy(au2_hbm.at[pl.ds(base, BPW)], au2v)
        pltpu.sync_copy(ai_hbm.at[pl.ds(base, BPW)], aiv)
        pltpu.sync_copy(ai2_hbm.at[pl.ds(base, BPW)], ai2v)
        cs = [
            pltpu.async_copy(mfu_hbm.at[uv], bmfu, sem),
            pltpu.async_copy(mfi_hbm.at[iv], bmfi, sem),
            pltpu.async_copy(mlpu_hbm.at[auv], bu0, sem),
            pltpu.async_copy(mlpu_hbm.at[au2v], bu1, sem),
            pltpu.async_copy(mlpi_hbm.at[aiv], bi0, sem),
            pltpu.async_copy(mlpi_hbm.at[ai2v], bi1, sem),
        ]
        for c in cs:
            c.wait()
        pltpu.sync_copy(bmfu, o_mfu.at[pl.ds(base, BPW)])
        pltpu.sync_copy(bmfi, o_mfi.at[pl.ds(base, BPW)])
        pltpu.sync_copy(bu0, o_u0.at[pl.ds(base, BPW)])
        pltpu.sync_copy(bu1, o_u1.at[pl.ds(base, BPW)])
        pltpu.sync_copy(bi0, o_i0.at[pl.ds(base, BPW)])
        pltpu.sync_copy(bi1, o_i1.at[pl.ds(base, BPW)])

    return _sc_gather


BM = 2048


def _realign(b0, b1, off):
    b = jnp.concatenate([b0, b1], axis=1)
    acc = b[:, 0:MLP_D]
    for k in range(1, 8):
        acc = jnp.where(off == 2 * k, b[:, 2 * k:2 * k + MLP_D], acc)
    return acc


def _tc_mlp_body(mfu, mfi, u0, u1, i0, i1, offu, offi,
                 W1a, W1b, b1, W2, b2, W3, b3,
                 Wl, bl, w2la, w2lb, b2l, out):
    f32 = jnp.float32
    xu = _realign(u0[...], u1[...], offu[...])
    xi = _realign(i0[...], i1[...], offi[...])
    x = (jnp.dot(xu, W1a[...], preferred_element_type=f32)
         + jnp.dot(xi, W1b[...], preferred_element_type=f32)
         + b1[...])
    x = jnp.maximum(x, 0.0)
    x = jnp.dot(x, W2[...], preferred_element_type=f32) + b2[...]
    x = jnp.maximum(x, 0.0)
    x = jnp.dot(x, W3[...], preferred_element_type=f32) + b3[...]
    x = jnp.maximum(x, 0.0)
    mlp_vec = jnp.dot(x, Wl[...], preferred_element_type=f32) + bl[...]
    mf_vec = mfu[...] * mfi[...]
    logit = (jnp.dot(mf_vec, w2la[...], preferred_element_type=f32)
             + jnp.dot(mlp_vec, w2lb[...], preferred_element_type=f32)
             + b2l[...])
    out[...] = jax.nn.sigmoid(logit)


def _tc_mlp(mfu, mfi, u0, u1, i0, i1, offu, offi,
            W1a, W1b, b1, W2, b2, W3, b3, Wl, bl, w2la, w2lb, b2l):
    def row_block(d):
        return pl.BlockSpec((BM, d), lambda m: (m, 0))

    def full(a):
        return pl.BlockSpec(a.shape, lambda m: (0,) * a.ndim)

    return pl.pallas_call(
        _tc_mlp_body,
        grid=(B // BM,),
        in_specs=[
            row_block(16), row_block(16), row_block(16), row_block(16),
            row_block(16), row_block(16), row_block(1), row_block(1),
            full(W1a), full(W1b), full(b1), full(W2), full(b2),
            full(W3), full(b3), full(Wl), full(bl),
            full(w2la), full(w2lb), full(b2l),
        ],
        out_specs=pl.BlockSpec((BM, 1), lambda m: (m, 0)),
        out_shape=jax.ShapeDtypeStruct((B, 1), jnp.float32),
    )(mfu, mfi, u0, u1, i0, i1, offu, offi,
      W1a, W1b, b1, W2, b2, W3, b3, Wl, bl, w2la, w2lb, b2l)


def kernel(inputs, mf_user, mf_item, mlp_user, mlp_item,
           W1, b1, W2, b2, W3, b3, Wl, bl, W2l, b2l):
    u = inputs[:, 0]
    i = inputs[:, 1]
    wu = u * MLP_D
    wi = i * MLP_D
    au = wu // 16
    ai = wi // 16
    au2 = jnp.minimum(au + 1, VROWS - 1)
    ai2 = jnp.minimum(ai + 1, VROWS - 1)
    offu = wu - au * 16
    offi = wi - ai * 16
    mfu, mfi, u0, u1, i0, i1 = _make_sc_gather()(
        u, i, au, au2, ai, ai2,
        mf_user, mf_item,
        mlp_user.reshape(VROWS, 16), mlp_item.reshape(VROWS, 16))
    return _tc_mlp(
        mfu, mfi, u0, u1, i0, i1,
        offu.reshape(B, 1), offi.reshape(B, 1),
        W1[:MLP_D], W1[MLP_D:], b1.reshape(1, -1),
        W2, b2.reshape(1, -1), W3, b3.reshape(1, -1),
        Wl, bl.reshape(1, -1),
        W2l[:MF_D], W2l[MF_D:], b2l.reshape(1, 1),
    )
